# baseline (speedup 1.0000x reference)
import functools
import jax
import jax.numpy as jnp
from jax import lax
from jax.experimental import pallas as pl
from jax.experimental.pallas import tpu as pltpu
from jax.experimental.pallas import tpu_sc as plsc

VOCAB = 1000000
EMBED_DIM = 64
BATCH = 4096
HIST = 50

NUM_CORES = 2
NUM_SUBCORES = 16
NW = NUM_CORES * NUM_SUBCORES
B_TOTAL = BATCH * HIST
B_PER_W = B_TOTAL // NW
CHUNK = 128
NCHUNK = B_PER_W // CHUNK
NBUF = 2

_mesh = plsc.VectorSubcoreMesh(
    core_axis_name="c", subcore_axis_name="s",
    num_cores=NUM_CORES, num_subcores=NUM_SUBCORES,
)


@functools.partial(
    pl.kernel,
    mesh=_mesh,
    out_type=jax.ShapeDtypeStruct((B_TOTAL, EMBED_DIM), jnp.float32),
    scratch_types=[
        pltpu.VMEM((NCHUNK, CHUNK), jnp.int32),
        [pltpu.VMEM((CHUNK, EMBED_DIM), jnp.float32) for _ in range(NBUF)],
        [pltpu.SemaphoreType.DMA for _ in range(NBUF)],
    ],
    compiler_params=pltpu.CompilerParams(use_tc_tiling_on_sc=False),
)
def _embed_gather(idx_hbm, table_hbm, out_hbm, idx_v, rows, gsems):
    wid = lax.axis_index("s") * NUM_CORES + lax.axis_index("c")
    base = wid * B_PER_W

    pltpu.sync_copy(idx_hbm.at[wid], idx_v)

    @pl.loop(0, NCHUNK, step=NBUF)
    def _body(j0):
        for b in range(NBUF):
            j = j0 + b
            pltpu.async_copy(table_hbm.at[idx_v.at[j]], rows[b], gsems[b])
        for b in range(NBUF):
            j = j0 + b
            pltpu.make_async_copy(table_hbm.at[idx_v.at[j]], rows[b],
                                  gsems[b]).wait()
            off = pl.multiple_of(base + j * CHUNK, CHUNK)
            pltpu.sync_copy(rows[b], out_hbm.at[pl.ds(off, CHUNK)])


def kernel(input, table):
    idx = input.reshape(NW, NCHUNK, CHUNK).astype(jnp.int32)
    out = _embed_gather(idx, table)
    return out.reshape(BATCH, HIST, EMBED_DIM)


# speedup vs baseline: 1.0162x; 1.0162x over previous
import functools
import jax
import jax.numpy as jnp
from jax import lax
from jax.experimental import pallas as pl
from jax.experimental.pallas import tpu as pltpu
from jax.experimental.pallas import tpu_sc as plsc

VOCAB = 1000000
EMBED_DIM = 64
BATCH = 4096
HIST = 50

NUM_CORES = 2
NUM_SUBCORES = 16
NW = NUM_CORES * NUM_SUBCORES
B_TOTAL = BATCH * HIST
B_PER_W = B_TOTAL // NW
CHUNK = 640
NCHUNK = B_PER_W // CHUNK
NBUF = 2

_mesh = plsc.VectorSubcoreMesh(
    core_axis_name="c", subcore_axis_name="s",
    num_cores=NUM_CORES, num_subcores=NUM_SUBCORES,
)


@functools.partial(
    pl.kernel,
    mesh=_mesh,
    out_type=jax.ShapeDtypeStruct((B_TOTAL, EMBED_DIM), jnp.float32),
    scratch_types=[
        pltpu.VMEM((NCHUNK, CHUNK), jnp.int32),
        [pltpu.VMEM((CHUNK, EMBED_DIM), jnp.float32) for _ in range(NBUF)],
        [pltpu.SemaphoreType.DMA for _ in range(NBUF)],
    ],
    compiler_params=pltpu.CompilerParams(use_tc_tiling_on_sc=False),
)
def _embed_gather(idx_hbm, table_hbm, out_hbm, idx_v, rows, gsems):
    wid = lax.axis_index("s") * NUM_CORES + lax.axis_index("c")
    base = wid * B_PER_W

    pltpu.sync_copy(idx_hbm.at[wid], idx_v)

    @pl.loop(0, NCHUNK, step=NBUF)
    def _body(j0):
        for b in range(NBUF):
            j = j0 + b
            pltpu.async_copy(table_hbm.at[idx_v.at[j]], rows[b], gsems[b])
        for b in range(NBUF):
            j = j0 + b
            pltpu.make_async_copy(table_hbm.at[idx_v.at[j]], rows[b],
                                  gsems[b]).wait()
            off = pl.multiple_of(base + j * CHUNK, CHUNK)
            pltpu.sync_copy(rows[b], out_hbm.at[pl.ds(off, CHUNK)])


def kernel(input, table):
    idx = input.reshape(NW, NCHUNK, CHUNK).astype(jnp.int32)
    out = _embed_gather(idx, table)
    return out.reshape(BATCH, HIST, EMBED_DIM)
